# baseline (speedup 1.0000x reference)
import functools

import jax
import jax.numpy as jnp
from jax import lax
from jax.experimental import pallas as pl
from jax.experimental.pallas import tpu as pltpu

_BATCH = 4096
_N_FIELDS = 26
_CARD = 200
_OUT_COLS = _N_FIELDS * _CARD
_LANES = 128
_NWIN = (_OUT_COLS + _LANES - 1) // _LANES
_R = 512
_SPLIT = 1536


def _make_body(out_dtype):
    def _body(x_ref, o_ref):
        y = x_ref[...] + _CARD * lax.broadcasted_iota(
            jnp.int32, (1, _N_FIELDS), 1)
        for j in range(_NWIN):
            lo = j * _LANES
            width = min(_LANES, _OUT_COLS - lo)
            i0 = lo // _CARD
            i1 = min(_N_FIELDS - 1, (lo + width - 1) // _CARD)
            c = lo + lax.broadcasted_iota(jnp.int32, (_R, width), 1)
            m = y[:, i0:i0 + 1] == c
            if i1 != i0:
                m = m | (y[:, i1:i1 + 1] == c)
            o_ref[:, lo:lo + width] = m if out_dtype == jnp.bool_ else (
                m.astype(out_dtype))
    return _body


def _onehot_part(x_part, out_dtype):
    rows = x_part.shape[0]
    return pl.pallas_call(
        _make_body(out_dtype),
        grid=(rows // _R,),
        in_specs=[pl.BlockSpec((_R, _N_FIELDS), lambda i: (i, 0))],
        out_specs=pl.BlockSpec((_R, _OUT_COLS), lambda i: (i, 0)),
        out_shape=jax.ShapeDtypeStruct((rows, _OUT_COLS), out_dtype),
        compiler_params=pltpu.CompilerParams(
            dimension_semantics=("arbitrary",)),
    )(x_part)


@jax.jit
def _onehot(x):
    out_b = _onehot_part(x[_SPLIT:], jnp.bool_).astype(jnp.int32)
    out_a = _onehot_part(x[:_SPLIT], jnp.int32)
    return jnp.concatenate([out_a, out_b], axis=0)


def kernel(x):
    return _onehot(x)


# speedup vs baseline: 1.8509x; 1.8509x over previous
import jax
import jax.numpy as jnp
from jax import lax
from jax.experimental import pallas as pl
from jax.experimental.pallas import tpu as pltpu

_BATCH = 4096
_N_FIELDS = 26
_CARD = 200
_OUT_COLS = _N_FIELDS * _CARD
_LANES = 128
_NWIN = (_OUT_COLS + _LANES - 1) // _LANES
_R = 512


def _body(x_ref, o_ref):
    y = x_ref[...] + _CARD * lax.broadcasted_iota(jnp.int32, (1, _N_FIELDS), 1)
    for j in range(_NWIN):
        lo = j * _LANES
        width = min(_LANES, _OUT_COLS - lo)
        i0 = lo // _CARD
        i1 = min(_N_FIELDS - 1, (lo + width - 1) // _CARD)
        c = lo + lax.broadcasted_iota(jnp.int32, (_R, width), 1)
        m = y[:, i0:i0 + 1] == c
        if i1 != i0:
            m = m | (y[:, i1:i1 + 1] == c)
        o_ref[:, lo:lo + width] = m.astype(jnp.int32)


@jax.jit
def _onehot_tc(x):
    return pl.pallas_call(
        _body,
        grid=(_BATCH // _R,),
        in_specs=[pl.BlockSpec((_R, _N_FIELDS), lambda i: (i, 0))],
        out_specs=pl.BlockSpec((_R, _OUT_COLS), lambda i: (i, 0)),
        out_shape=jax.ShapeDtypeStruct((_BATCH, _OUT_COLS), jnp.int32),
        compiler_params=pltpu.CompilerParams(
            dimension_semantics=("arbitrary",)),
    )(x)


def kernel(x):
    return _onehot_tc(x)
